# baseline (speedup 1.0000x reference)
import functools

import jax
import jax.numpy as jnp
from jax import lax
from jax.experimental import pallas as pl
from jax.experimental.pallas import tpu as pltpu
from jax.experimental.pallas import tpu_sc as plsc

BATCH = 1024
SEQ = 200
D = 256
V = 1000
B = BATCH * SEQ
NC = 2
NS = 16
NW = NC * NS
B_PER_W = B // NW
C = 128
NCHUNK = B_PER_W // C
NBUF = 3


def _make_gather():
    mesh = plsc.VectorSubcoreMesh(core_axis_name="c", subcore_axis_name="s")
    scratch = [
        pltpu.VMEM((B_PER_W + 16,), jnp.int32),
        pltpu.VMEM((NBUF, C, D), jnp.float32),
        pltpu.VMEM_SHARED((V, D), jnp.float32),
    ]
    scratch += [pltpu.SemaphoreType.DMA] * NBUF
    scratch += [pltpu.SemaphoreType.DMA] * NBUF

    @functools.partial(
        pl.kernel,
        mesh=mesh,
        out_type=jax.ShapeDtypeStruct((B, D), jnp.float32),
        scratch_types=scratch,
    )
    def gather_kernel(tbl_hbm, idx_hbm, out_hbm,
                      idx_v, rows_v, tbl_sh, *sems):
        gsem = sems[:NBUF]
        ssem = sems[NBUF:]
        sid = lax.axis_index("s")
        wid = sid * NC + lax.axis_index("c")
        base = wid * B_PER_W

        @pl.when(sid < NS - 1)
        def _():
            pltpu.sync_copy(tbl_hbm.at[pl.ds(sid * 64, 64)],
                            tbl_sh.at[pl.ds(sid * 64, 64)])

        @pl.when(sid == NS - 1)
        def _():
            pltpu.sync_copy(tbl_hbm.at[pl.ds(960, 40)],
                            tbl_sh.at[pl.ds(960, 40)])

        pltpu.sync_copy(idx_hbm.at[wid], idx_v.at[pl.ds(0, B_PER_W)])
        plsc.subcore_barrier()

        def fill_rows(j, b):
            @plsc.parallel_loop(0, C // 16, unroll=2)
            def _(g):
                i0 = g * 16
                v = idx_v[pl.ds(j * C + i0, 16)]
                for k in range(16):
                    pltpu.make_async_copy(
                        tbl_sh.at[pl.ds(v[k], 1)],
                        rows_v.at[b].at[pl.ds(i0 + k, 1)],
                        gsem[b],
                    ).start()

        def wait_rows(b):
            pltpu.make_async_copy(
                tbl_hbm.at[pl.ds(0, C)], rows_v.at[b], gsem[b]
            ).wait()

        def start_store(j, b):
            pltpu.make_async_copy(
                rows_v.at[b], out_hbm.at[pl.ds(base + j * C, C)], ssem[b]
            ).start()

        def wait_store(b):
            pltpu.make_async_copy(
                rows_v.at[b], out_hbm.at[pl.ds(base, C)], ssem[b]
            ).wait()

        fill_rows(0, 0)

        @pl.loop(0, NCHUNK - 2, step=NBUF)
        def _(i):
            for b in range(NBUF):
                j = i + b
                bp = (b + 1) % NBUF

                @pl.when(j >= 2)
                def _():
                    wait_store(bp)

                fill_rows(j + 1, bp)
                wait_rows(b)
                start_store(j, b)

        j0 = NCHUNK - 2
        b0 = j0 % NBUF
        bp0 = (b0 + 1) % NBUF
        wait_store(bp0)
        fill_rows(j0 + 1, bp0)
        wait_rows(b0)
        start_store(j0, b0)

        j1 = NCHUNK - 1
        b1 = j1 % NBUF
        wait_store((b1 + 1) % NBUF)
        wait_rows(b1)
        start_store(j1, b1)

        wait_store(b0)
        wait_store(b1)

    return gather_kernel


_gather = _make_gather()


def kernel(inp, sincos_table, translation_bias):
    tbl = sincos_table + translation_bias[None, :].astype(sincos_table.dtype)
    idx = inp.reshape(NW, B_PER_W).astype(jnp.int32)
    out = _gather(tbl, idx)
    return out.reshape(BATCH, SEQ, D)


# speedup vs baseline: 1.0224x; 1.0224x over previous
import functools

import jax
import jax.numpy as jnp
from jax import lax
from jax.experimental import pallas as pl
from jax.experimental.pallas import tpu as pltpu
from jax.experimental.pallas import tpu_sc as plsc

BATCH = 1024
SEQ = 200
D = 256
V = 1000
B = BATCH * SEQ
NC = 2
NS = 16
NW = NC * NS
B_PER_W = B // NW
C = 64
NCHUNK = B_PER_W // C
NBUF = 6


def _make_gather():
    mesh = plsc.VectorSubcoreMesh(core_axis_name="c", subcore_axis_name="s")
    scratch = [
        pltpu.VMEM((B_PER_W + 16,), jnp.int32),
        pltpu.VMEM((NBUF, C, D), jnp.float32),
        pltpu.VMEM_SHARED((V, D), jnp.float32),
    ]
    scratch += [pltpu.SemaphoreType.DMA] * NBUF
    scratch += [pltpu.SemaphoreType.DMA] * NBUF

    @functools.partial(
        pl.kernel,
        mesh=mesh,
        out_type=jax.ShapeDtypeStruct((B, D), jnp.float32),
        scratch_types=scratch,
    )
    def gather_kernel(tbl_hbm, idx_hbm, out_hbm,
                      idx_v, rows_v, tbl_sh, *sems):
        gsem = sems[:NBUF]
        ssem = sems[NBUF:]
        sid = lax.axis_index("s")
        wid = sid * NC + lax.axis_index("c")
        base = wid * B_PER_W

        @pl.when(sid < NS - 1)
        def _():
            pltpu.sync_copy(tbl_hbm.at[pl.ds(sid * 64, 64)],
                            tbl_sh.at[pl.ds(sid * 64, 64)])

        @pl.when(sid == NS - 1)
        def _():
            pltpu.sync_copy(tbl_hbm.at[pl.ds(960, 40)],
                            tbl_sh.at[pl.ds(960, 40)])

        pltpu.sync_copy(idx_hbm.at[wid], idx_v.at[pl.ds(0, B_PER_W)])
        plsc.subcore_barrier()

        def fill_rows(j, b):
            @pl.loop(0, C // 16)
            def _(g):
                i0 = g * 16
                v = idx_v[pl.ds(j * C + i0, 16)]
                for k in range(16):
                    pltpu.make_async_copy(
                        tbl_sh.at[pl.ds(v[k], 1)],
                        rows_v.at[b].at[pl.ds(i0 + k, 1)],
                        gsem[b],
                    ).start()

        def wait_rows(b):
            pltpu.make_async_copy(
                tbl_hbm.at[pl.ds(0, C)], rows_v.at[b], gsem[b]
            ).wait()

        def start_store(j, b):
            pltpu.make_async_copy(
                rows_v.at[b], out_hbm.at[pl.ds(base + j * C, C)], ssem[b]
            ).start()

        def wait_store(b):
            pltpu.make_async_copy(
                rows_v.at[b], out_hbm.at[pl.ds(base, C)], ssem[b]
            ).wait()

        fill_rows(0, 0)

        NMAIN = ((NCHUNK - 2) // NBUF) * NBUF

        @pl.loop(0, NMAIN, step=NBUF)
        def _(i):
            for b in range(NBUF):
                j = i + b
                bp = (b + 1) % NBUF

                @pl.when(j >= NBUF - 1)
                def _():
                    wait_store(bp)

                fill_rows(j + 1, bp)
                wait_rows(b)
                start_store(j, b)

        for j in range(NMAIN, NCHUNK - 1):
            b = j % NBUF
            bp = (b + 1) % NBUF
            wait_store(bp)
            fill_rows(j + 1, bp)
            wait_rows(b)
            start_store(j, b)

        j1 = NCHUNK - 1
        wait_rows(j1 % NBUF)
        start_store(j1, j1 % NBUF)

        for j in range(NCHUNK - NBUF, NCHUNK):
            wait_store(j % NBUF)

    return gather_kernel


_gather = _make_gather()


def kernel(inp, sincos_table, translation_bias):
    tbl = sincos_table + translation_bias[None, :].astype(sincos_table.dtype)
    idx = inp.reshape(NW, B_PER_W).astype(jnp.int32)
    out = _gather(tbl, idx)
    return out.reshape(BATCH, SEQ, D)
